# baseline (speedup 1.0000x reference)
import jax
import jax.numpy as jnp
from jax import lax
from jax.experimental import pallas as pl
from jax.experimental.pallas import tpu as pltpu
from jax.experimental.pallas import tpu_sc as plsc

SEQ = 4096
EMB = 128
BATCH = 4
NUM_INPUTS = 2
NUM_CORES = 2
NUM_SUBCORES = 16
NW = NUM_CORES * NUM_SUBCORES
ROWS = SEQ // NW
LANES = 16
GROUPS = EMB // LANES


def _body(x0, x1, upw, lpw, rtw, out,
          pos0, pos1, xb0, xb1, ob0, ob1, lp_v, rot_v,
          s_in0, s_in1, s_out0, s_out1, s_small):
    cid = lax.axis_index("c")
    sid = lax.axis_index("s")
    wid = sid * NUM_CORES + cid
    seq0 = wid * ROWS

    pltpu.async_copy(lpw, lp_v, s_small).wait()
    pltpu.async_copy(rtw, rot_v, s_small).wait()
    pltpu.async_copy(upw.at[pl.ds(seq0, ROWS)], pos0, s_small).wait()

    def fold(r, _):
        for g in range(GROUPS):
            col = pl.ds(g * LANES, LANES)
            u = pos0[r, col]
            pos1[r, col] = u + lp_v[1, col]
            pos0[r, col] = u + lp_v[0, col]
        return _
    lax.fori_loop(0, ROWS, fold, 0, unroll=2)

    @pl.when(wid == 0)
    def _():
        for b in range(BATCH):
            pltpu.async_copy(rot_v, out.at[b, pl.ds(0, 1)], s_small).wait()

    xsrc = (x0, x1)
    xbufs = (xb0, xb1)
    obufs = (ob0, ob1)
    in_sems = (s_in0, s_in1)
    out_sems = (s_out0, s_out1)
    pos = (pos0, pos1)

    def chunk(k):
        b, j = k // NUM_INPUTS, k % NUM_INPUTS
        return b, j

    def start_in(k):
        b, j = chunk(k)
        return pltpu.async_copy(xsrc[j].at[b, pl.ds(seq0, ROWS)],
                                xbufs[k % 2], in_sems[k % 2])

    def start_out(k):
        b, j = chunk(k)
        dst = out.at[b, pl.ds(1 + j * SEQ + seq0, ROWS)]
        return pltpu.async_copy(obufs[k % 2], dst, out_sems[k % 2])

    pending_in = {0: start_in(0)}
    pending_out = {}
    for k in range(BATCH * NUM_INPUTS):
        if k + 1 < BATCH * NUM_INPUTS:
            pending_in[k + 1] = start_in(k + 1)
        pending_in.pop(k).wait()
        if k - 2 in pending_out:
            pending_out.pop(k - 2).wait()
        _, j = chunk(k)
        xb, ob, pj = xbufs[k % 2], obufs[k % 2], pos[j]

        def add(r, _):
            for g in range(GROUPS):
                col = pl.ds(g * LANES, LANES)
                ob[r, col] = xb[r, col] + pj[r, col]
            return _
        lax.fori_loop(0, ROWS, add, 0, unroll=2)
        pending_out[k] = start_out(k)
    for k in sorted(pending_out):
        pending_out.pop(k).wait()


def kernel(x0, x1, unique_pos_w, layer_pos_w, rot_token_w):
    mesh = plsc.VectorSubcoreMesh(core_axis_name="c", subcore_axis_name="s")
    f32 = jnp.float32
    run = pl.kernel(
        _body,
        out_type=jax.ShapeDtypeStruct((BATCH, NUM_INPUTS * SEQ + 1, EMB), f32),
        mesh=mesh,
        scratch_types=[
            pltpu.VMEM((ROWS, EMB), f32),
            pltpu.VMEM((ROWS, EMB), f32),
            pltpu.VMEM((ROWS, EMB), f32),
            pltpu.VMEM((ROWS, EMB), f32),
            pltpu.VMEM((ROWS, EMB), f32),
            pltpu.VMEM((ROWS, EMB), f32),
            pltpu.VMEM((NUM_INPUTS, EMB), f32),
            pltpu.VMEM((1, EMB), f32),
            pltpu.SemaphoreType.DMA,
            pltpu.SemaphoreType.DMA,
            pltpu.SemaphoreType.DMA,
            pltpu.SemaphoreType.DMA,
            pltpu.SemaphoreType.DMA,
        ],
        compiler_params=pltpu.CompilerParams(use_tc_tiling_on_sc=False),
    )
    return run(x0, x1, unique_pos_w, layer_pos_w, rot_token_w)


# speedup vs baseline: 1.8653x; 1.8653x over previous
import jax
import jax.numpy as jnp
from jax import lax
from jax.experimental import pallas as pl
from jax.experimental.pallas import tpu as pltpu
from jax.experimental.pallas import tpu_sc as plsc

SEQ = 4096
EMB = 128
BATCH = 4
NUM_INPUTS = 2
NUM_CORES = 2
NUM_SUBCORES = 16
NW = NUM_CORES * NUM_SUBCORES
SLAB = 128
XROWS = SLAB + 8
PROWS = 2 * SLAB + 8
LANES = 16
GROUPS = EMB // LANES
NCHUNK = BATCH * NUM_INPUTS


def _body(x0, x1, upw, lpw, rtw, out,
          pb, xb0, xb1, ob0, ob1, lp_v, rot_v,
          s_in0, s_in1, s_out0, s_out1, s_small):
    cid = lax.axis_index("c")
    sid = lax.axis_index("s")
    w = sid * NUM_CORES + cid
    a0 = pl.multiple_of(w * (2 * SLAB), 2 * SLAB)

    d_lp = pltpu.async_copy(lpw, lp_v, s_small)
    d_rt = pltpu.async_copy(rtw, rot_v, s_small)

    @pl.when(w <= 15)
    def _():
        base = pl.multiple_of(jnp.maximum(a0 - 8, 0), 8)
        pltpu.async_copy(upw.at[pl.ds(base, PROWS)], pb, s_small)

    @pl.when(w == 16)
    def _():
        pltpu.async_copy(upw.at[pl.ds(SEQ - 8, 8)], pb.at[pl.ds(0, 8)], s_small)
        pltpu.async_copy(upw.at[pl.ds(0, PROWS - 8)], pb.at[pl.ds(8, PROWS - 8)], s_small)

    @pl.when(w >= 17)
    def _():
        base = pl.multiple_of(a0 - SEQ - 8, 8)
        pltpu.async_copy(upw.at[pl.ds(base, PROWS)], pb, s_small)

    d_lp.wait()
    d_rt.wait()
    pltpu.make_async_copy(upw.at[pl.ds(0, PROWS)], pb, s_small).wait()

    cut = jnp.where(w < 16, PROWS, jnp.where(w == 16, 8, 0))
    lp0 = [lp_v[0, pl.ds(g * LANES, LANES)] for g in range(GROUPS)]
    lp1 = [lp_v[1, pl.ds(g * LANES, LANES)] for g in range(GROUPS)]

    def fold0(r, c):
        for g in range(GROUPS):
            col = pl.ds(g * LANES, LANES)
            pb[r, col] = pb[r, col] + lp0[g]
        return c

    def fold1(r, c):
        for g in range(GROUPS):
            col = pl.ds(g * LANES, LANES)
            pb[r, col] = pb[r, col] + lp1[g]
        return c

    lax.fori_loop(0, cut, fold0, 0)
    lax.fori_loop(cut, PROWS, fold1, 0)

    pshift = jnp.where(w == 0, -1, 7)

    xbufs = (xb0, xb1)
    obufs = (ob0, ob1)
    in_sems = (s_in0, s_in1)
    out_sems = (s_out0, s_out1)

    def start_in(c):
        b, h = c // NUM_INPUTS, c % NUM_INPUTS
        xb, sem = xbufs[c % 2], in_sems[c % 2]
        ah = pl.multiple_of(a0 + h * SLAB, SLAB)
        if h == 0:
            @pl.when(w <= 15)
            def _():
                base = pl.multiple_of(jnp.maximum(ah - 8, 0), 8)
                pltpu.async_copy(x0.at[b, pl.ds(base, XROWS)], xb, sem)

            @pl.when(w == 16)
            def _():
                pltpu.async_copy(x0.at[b, pl.ds(SEQ - 8, 8)], xb.at[pl.ds(0, 8)], sem)
                pltpu.async_copy(x1.at[b, pl.ds(0, SLAB)], xb.at[pl.ds(8, SLAB)], sem)

            @pl.when(w >= 17)
            def _():
                base = pl.multiple_of(ah - SEQ - 8, 8)
                pltpu.async_copy(x1.at[b, pl.ds(base, XROWS)], xb, sem)
        else:
            @pl.when(w <= 15)
            def _():
                base = pl.multiple_of(ah - 8, 8)
                pltpu.async_copy(x0.at[b, pl.ds(base, XROWS)], xb, sem)

            @pl.when(w >= 16)
            def _():
                base = pl.multiple_of(ah - SEQ - 8, 8)
                pltpu.async_copy(x1.at[b, pl.ds(base, XROWS)], xb, sem)

    def wait_in(c):
        pltpu.make_async_copy(x0.at[0, pl.ds(0, XROWS)],
                              xbufs[c % 2], in_sems[c % 2]).wait()

    start_in(0)
    pending_out = {}
    for c in range(NCHUNK):
        b, h = c // NUM_INPUTS, c % NUM_INPUTS
        if c + 1 < NCHUNK:
            start_in(c + 1)
        wait_in(c)
        if c - 2 in pending_out:
            pending_out.pop(c - 2).wait()
        xb, ob = xbufs[c % 2], obufs[c % 2]

        xshift = pshift if h == 0 else 7
        poff = h * SLAB + pshift

        def add(r, acc):
            xi = jnp.maximum(r + xshift, 0)
            pi = jnp.maximum(r + poff, 0)
            for g in range(GROUPS):
                col = pl.ds(g * LANES, LANES)
                ob[r, col] = xb[xi, col] + pb[pi, col]
            return acc

        lax.fori_loop(0, SLAB, add, 0, unroll=4)

        if h == 0:
            @pl.when(w == 0)
            def _():
                for g in range(GROUPS):
                    col = pl.ds(g * LANES, LANES)
                    ob[0, col] = rot_v[0, col]

        ah = pl.multiple_of(a0 + h * SLAB, SLAB)
        pending_out[c] = pltpu.async_copy(
            ob.at[pl.ds(0, SLAB)], out.at[b, pl.ds(ah, SLAB)], out_sems[c % 2])

        if h == 1:
            @pl.when(w == NW - 1)
            def _():
                for g in range(GROUPS):
                    col = pl.ds(g * LANES, LANES)
                    ob[SLAB, col] = xb[XROWS - 1, col] + pb[PROWS - 1, col]
                pltpu.async_copy(ob.at[pl.ds(SLAB, 1)],
                                 out.at[b, pl.ds(NUM_INPUTS * SEQ, 1)], s_small)
                pltpu.make_async_copy(ob.at[pl.ds(SLAB, 1)],
                                      out.at[b, pl.ds(NUM_INPUTS * SEQ, 1)],
                                      s_small).wait()

    for c in sorted(pending_out):
        pending_out.pop(c).wait()


def kernel(x0, x1, unique_pos_w, layer_pos_w, rot_token_w):
    mesh = plsc.VectorSubcoreMesh(core_axis_name="c", subcore_axis_name="s")
    f32 = jnp.float32
    run = pl.kernel(
        _body,
        out_type=jax.ShapeDtypeStruct((BATCH, NUM_INPUTS * SEQ + 1, EMB), f32),
        mesh=mesh,
        scratch_types=[
            pltpu.VMEM((PROWS, EMB), f32),
            pltpu.VMEM((XROWS, EMB), f32),
            pltpu.VMEM((XROWS, EMB), f32),
            pltpu.VMEM((SLAB + 1, EMB), f32),
            pltpu.VMEM((SLAB + 1, EMB), f32),
            pltpu.VMEM((NUM_INPUTS, EMB), f32),
            pltpu.VMEM((1, EMB), f32),
            pltpu.SemaphoreType.DMA,
            pltpu.SemaphoreType.DMA,
            pltpu.SemaphoreType.DMA,
            pltpu.SemaphoreType.DMA,
            pltpu.SemaphoreType.DMA,
        ],
    )
    return run(x0, x1, unique_pos_w, layer_pos_w, rot_token_w)


# speedup vs baseline: 2.6953x; 1.4449x over previous
import jax
import jax.numpy as jnp
from jax import lax
from jax.experimental import pallas as pl
from jax.experimental.pallas import tpu as pltpu
from jax.experimental.pallas import tpu_sc as plsc

SEQ = 4096
EMB = 128
BATCH = 4
NUM_INPUTS = 2
NUM_CORES = 2
NUM_SUBCORES = 16
NW = NUM_CORES * NUM_SUBCORES
WROWS = 2 * SEQ // NW
CH = 32
NCHUNK = WROWS // CH
XROWS = CH + 8
PROWS = WROWS + 8
LANES = 16
GROUPS = EMB // LANES


def _body(x0, x1, upw, lpw, rtw, out,
          pb, xb0, xb1, ob0, ob1, rb, lp_v, rot_v,
          s_in0, s_in1, s_out0, s_out1, s_small):
    cid = lax.axis_index("c")
    sid = lax.axis_index("s")
    w = sid * NUM_CORES + cid
    a0 = pl.multiple_of(w * WROWS, WROWS)

    d_lp = pltpu.async_copy(lpw, lp_v, s_small)
    d_rt = pltpu.async_copy(rtw, rot_v, s_small)

    @pl.when(w <= 15)
    def _():
        base = pl.multiple_of(jnp.maximum(a0 - 8, 0), 8)
        pltpu.async_copy(upw.at[pl.ds(base, PROWS)], pb, s_small)

    @pl.when(w == 16)
    def _():
        pltpu.async_copy(upw.at[pl.ds(SEQ - 8, 8)], pb.at[pl.ds(0, 8)], s_small)
        pltpu.async_copy(upw.at[pl.ds(0, PROWS - 8)], pb.at[pl.ds(8, PROWS - 8)], s_small)

    @pl.when(w >= 17)
    def _():
        base = pl.multiple_of(a0 - SEQ - 8, 8)
        pltpu.async_copy(upw.at[pl.ds(base, PROWS)], pb, s_small)

    d_lp.wait()
    d_rt.wait()
    pltpu.make_async_copy(upw.at[pl.ds(0, PROWS)], pb, s_small).wait()

    cut = jnp.where(w < 16, PROWS, jnp.where(w == 16, 8, 0))
    lp0 = [lp_v[0, pl.ds(g * LANES, LANES)] for g in range(GROUPS)]
    lp1 = [lp_v[1, pl.ds(g * LANES, LANES)] for g in range(GROUPS)]

    def fold0(r, c):
        for g in range(GROUPS):
            col = pl.ds(g * LANES, LANES)
            pb[r, col] = pb[r, col] + lp0[g]
        return c

    def fold1(r, c):
        for g in range(GROUPS):
            col = pl.ds(g * LANES, LANES)
            pb[r, col] = pb[r, col] + lp1[g]
        return c

    lax.fori_loop(0, cut, fold0, 0)
    lax.fori_loop(cut, PROWS, fold1, 0)

    pshift = jnp.where(w == 0, -1, 7)

    xbufs = (xb0, xb1)
    obufs = (ob0, ob1)
    in_sems = (s_in0, s_in1)
    out_sems = (s_out0, s_out1)

    def start_in(h):
        xb, sem = xbufs[h % 2], in_sems[h % 2]
        ah = pl.multiple_of(a0 + h * CH, CH)
        if h == 0:
            @pl.when(w <= 15)
            def _():
                base = pl.multiple_of(jnp.maximum(ah - 8, 0), 8)
                pltpu.async_copy(x0.at[:, pl.ds(base, XROWS)], xb, sem)

            @pl.when(w == 16)
            def _():
                pltpu.async_copy(x0.at[:, pl.ds(SEQ - 8, 8)], xb.at[:, pl.ds(0, 8)], sem)
                pltpu.async_copy(x1.at[:, pl.ds(0, CH)], xb.at[:, pl.ds(8, CH)], sem)

            @pl.when(w >= 17)
            def _():
                base = pl.multiple_of(ah - SEQ - 8, 8)
                pltpu.async_copy(x1.at[:, pl.ds(base, XROWS)], xb, sem)
        else:
            @pl.when(w <= 15)
            def _():
                base = pl.multiple_of(ah - 8, 8)
                pltpu.async_copy(x0.at[:, pl.ds(base, XROWS)], xb, sem)

            @pl.when(w >= 16)
            def _():
                base = pl.multiple_of(ah - SEQ - 8, 8)
                pltpu.async_copy(x1.at[:, pl.ds(base, XROWS)], xb, sem)

    def wait_in(h):
        pltpu.make_async_copy(x0.at[:, pl.ds(0, XROWS)],
                              xbufs[h % 2], in_sems[h % 2]).wait()

    start_in(0)
    pending_out = {}
    for h in range(NCHUNK):
        if h + 1 < NCHUNK:
            start_in(h + 1)
        wait_in(h)
        if h - 2 in pending_out:
            pending_out.pop(h - 2).wait()
        xb, ob = xbufs[h % 2], obufs[h % 2]

        xshift = pshift if h == 0 else 7
        poff = h * CH + pshift

        def add(r, acc):
            xi = jnp.maximum(r + xshift, 0)
            pi = jnp.maximum(r + poff, 0)
            for g in range(GROUPS):
                col = pl.ds(g * LANES, LANES)
                pv = pb[pi, col]
                for b in range(BATCH):
                    ob[r, b, col] = xb[b, xi, col] + pv
            return acc

        lax.fori_loop(0, CH, add, 0, unroll=2)

        if h == 0:
            @pl.when(w == 0)
            def _():
                for b in range(BATCH):
                    for g in range(GROUPS):
                        col = pl.ds(g * LANES, LANES)
                        ob[0, b, col] = rot_v[0, col]

        ah = pl.multiple_of(a0 + h * CH, CH)
        pending_out[h] = pltpu.async_copy(ob, out.at[pl.ds(ah, CH)],
                                          out_sems[h % 2])

        if h == NCHUNK - 1:
            @pl.when(w == NW - 1)
            def _():
                for g in range(GROUPS):
                    col = pl.ds(g * LANES, LANES)
                    pv = pb[PROWS - 1, col]
                    for b in range(BATCH):
                        rb[0, b, col] = xb[b, XROWS - 1, col] + pv
                pltpu.async_copy(rb, out.at[pl.ds(NUM_INPUTS * SEQ, 1)], s_small)
                pltpu.make_async_copy(rb, out.at[pl.ds(NUM_INPUTS * SEQ, 1)],
                                      s_small).wait()

    for h in sorted(pending_out):
        pending_out.pop(h).wait()


def kernel(x0, x1, unique_pos_w, layer_pos_w, rot_token_w):
    mesh = plsc.VectorSubcoreMesh(core_axis_name="c", subcore_axis_name="s")
    f32 = jnp.float32
    run = pl.kernel(
        _body,
        out_type=jax.ShapeDtypeStruct((NUM_INPUTS * SEQ + 1, BATCH, EMB), f32),
        mesh=mesh,
        scratch_types=[
            pltpu.VMEM((PROWS, EMB), f32),
            pltpu.VMEM((BATCH, XROWS, EMB), f32),
            pltpu.VMEM((BATCH, XROWS, EMB), f32),
            pltpu.VMEM((CH, BATCH, EMB), f32),
            pltpu.VMEM((CH, BATCH, EMB), f32),
            pltpu.VMEM((1, BATCH, EMB), f32),
            pltpu.VMEM((NUM_INPUTS, EMB), f32),
            pltpu.VMEM((1, EMB), f32),
            pltpu.SemaphoreType.DMA,
            pltpu.SemaphoreType.DMA,
            pltpu.SemaphoreType.DMA,
            pltpu.SemaphoreType.DMA,
            pltpu.SemaphoreType.DMA,
        ],
    )
    p = run(x0, x1, unique_pos_w, layer_pos_w, rot_token_w)
    return jnp.transpose(p, (1, 0, 2))


# speedup vs baseline: 3.7775x; 1.4015x over previous
import jax
import jax.numpy as jnp
from jax import lax
from jax.experimental import pallas as pl
from jax.experimental.pallas import tpu as pltpu
from jax.experimental.pallas import tpu_sc as plsc

SEQ = 4096
EMB = 128
BATCH = 4
NUM_INPUTS = 2
NUM_CORES = 2
NUM_SUBCORES = 16
NW = NUM_CORES * NUM_SUBCORES
WROWS = 2 * SEQ // NW
CH = 32
NCHUNK = WROWS // CH
XROWS = CH + 8
PROWS = WROWS + 8
LANES = 16
GROUPS = EMB // LANES


def _body(x0, x1, upw, lpw, rtw, out,
          pb, xb0, xb1, ob0, ob1, rb, lp_v, rot_v,
          s_in0, s_in1, s_out0, s_out1, s_small):
    cid = lax.axis_index("c")
    sid = lax.axis_index("s")
    w = sid * NUM_CORES + cid
    a0 = pl.multiple_of(w * WROWS, WROWS)

    d_lp = pltpu.async_copy(lpw, lp_v, s_small)
    d_rt = pltpu.async_copy(rtw, rot_v, s_small)

    @pl.when(w <= 15)
    def _():
        base = pl.multiple_of(jnp.maximum(a0 - 8, 0), 8)
        pltpu.async_copy(upw.at[pl.ds(base, PROWS)], pb, s_small)

    @pl.when(w == 16)
    def _():
        pltpu.async_copy(upw.at[pl.ds(SEQ - 8, 8)], pb.at[pl.ds(0, 8)], s_small)
        pltpu.async_copy(upw.at[pl.ds(0, PROWS - 8)], pb.at[pl.ds(8, PROWS - 8)], s_small)

    @pl.when(w >= 17)
    def _():
        base = pl.multiple_of(a0 - SEQ - 8, 8)
        pltpu.async_copy(upw.at[pl.ds(base, PROWS)], pb, s_small)

    d_lp.wait()
    d_rt.wait()
    pltpu.make_async_copy(upw.at[pl.ds(0, PROWS)], pb, s_small).wait()

    cut = jnp.where(w < 16, PROWS, jnp.where(w == 16, 8, 0))
    lp0 = [lp_v[0, pl.ds(g * LANES, LANES)] for g in range(GROUPS)]
    lp1 = [lp_v[1, pl.ds(g * LANES, LANES)] for g in range(GROUPS)]

    @plsc.parallel_loop(0, cut, unroll=2)
    def _(r):
        for g in range(GROUPS):
            col = pl.ds(g * LANES, LANES)
            pb[r, col] = pb[r, col] + lp0[g]

    @plsc.parallel_loop(cut, PROWS, unroll=2)
    def _(r):
        for g in range(GROUPS):
            col = pl.ds(g * LANES, LANES)
            pb[r, col] = pb[r, col] + lp1[g]

    pshift = jnp.where(w == 0, -1, 7)

    xbufs = (xb0, xb1)
    obufs = (ob0, ob1)
    in_sems = (s_in0, s_in1)
    out_sems = (s_out0, s_out1)

    def start_in(h):
        xb, sem = xbufs[h % 2], in_sems[h % 2]
        ah = pl.multiple_of(a0 + h * CH, CH)
        if h == 0:
            @pl.when(w <= 15)
            def _():
                base = pl.multiple_of(jnp.maximum(ah - 8, 0), 8)
                pltpu.async_copy(x0.at[:, pl.ds(base, XROWS)], xb, sem)

            @pl.when(w == 16)
            def _():
                pltpu.async_copy(x0.at[:, pl.ds(SEQ - 8, 8)], xb.at[:, pl.ds(0, 8)], sem)
                pltpu.async_copy(x1.at[:, pl.ds(0, CH)], xb.at[:, pl.ds(8, CH)], sem)

            @pl.when(w >= 17)
            def _():
                base = pl.multiple_of(ah - SEQ - 8, 8)
                pltpu.async_copy(x1.at[:, pl.ds(base, XROWS)], xb, sem)
        else:
            @pl.when(w <= 15)
            def _():
                base = pl.multiple_of(ah - 8, 8)
                pltpu.async_copy(x0.at[:, pl.ds(base, XROWS)], xb, sem)

            @pl.when(w >= 16)
            def _():
                base = pl.multiple_of(ah - SEQ - 8, 8)
                pltpu.async_copy(x1.at[:, pl.ds(base, XROWS)], xb, sem)

    def wait_in(h):
        pltpu.make_async_copy(x0.at[:, pl.ds(0, XROWS)],
                              xbufs[h % 2], in_sems[h % 2]).wait()

    start_in(0)
    pending_out = {}
    for h in range(NCHUNK):
        if h + 1 < NCHUNK:
            start_in(h + 1)
        wait_in(h)
        if h - 2 in pending_out:
            pending_out.pop(h - 2).wait()
        xb, ob = xbufs[h % 2], obufs[h % 2]

        xshift = pshift if h == 0 else 7
        poff = h * CH + pshift

        @plsc.parallel_loop(0, CH, unroll=4)
        def _(r):
            xi = jnp.maximum(r + xshift, 0)
            pi = jnp.maximum(r + poff, 0)
            for g in range(GROUPS):
                col = pl.ds(g * LANES, LANES)
                pv = pb[pi, col]
                for b in range(BATCH):
                    ob[r, b, col] = xb[b, xi, col] + pv

        if h == 0:
            @pl.when(w == 0)
            def _():
                for b in range(BATCH):
                    for g in range(GROUPS):
                        col = pl.ds(g * LANES, LANES)
                        ob[0, b, col] = rot_v[0, col]

        ah = pl.multiple_of(a0 + h * CH, CH)
        pending_out[h] = pltpu.async_copy(ob, out.at[pl.ds(ah, CH)],
                                          out_sems[h % 2])

        if h == NCHUNK - 1:
            @pl.when(w == NW - 1)
            def _():
                for g in range(GROUPS):
                    col = pl.ds(g * LANES, LANES)
                    pv = pb[PROWS - 1, col]
                    for b in range(BATCH):
                        rb[0, b, col] = xb[b, XROWS - 1, col] + pv
                pltpu.async_copy(rb, out.at[pl.ds(NUM_INPUTS * SEQ, 1)], s_small)
                pltpu.make_async_copy(rb, out.at[pl.ds(NUM_INPUTS * SEQ, 1)],
                                      s_small).wait()

    for h in sorted(pending_out):
        pending_out.pop(h).wait()


def kernel(x0, x1, unique_pos_w, layer_pos_w, rot_token_w):
    mesh = plsc.VectorSubcoreMesh(core_axis_name="c", subcore_axis_name="s")
    f32 = jnp.float32
    run = pl.kernel(
        _body,
        out_type=jax.ShapeDtypeStruct((NUM_INPUTS * SEQ + 1, BATCH, EMB), f32),
        mesh=mesh,
        scratch_types=[
            pltpu.VMEM((PROWS, EMB), f32),
            pltpu.VMEM((BATCH, XROWS, EMB), f32),
            pltpu.VMEM((BATCH, XROWS, EMB), f32),
            pltpu.VMEM((CH, BATCH, EMB), f32),
            pltpu.VMEM((CH, BATCH, EMB), f32),
            pltpu.VMEM((1, BATCH, EMB), f32),
            pltpu.VMEM((NUM_INPUTS, EMB), f32),
            pltpu.VMEM((1, EMB), f32),
            pltpu.SemaphoreType.DMA,
            pltpu.SemaphoreType.DMA,
            pltpu.SemaphoreType.DMA,
            pltpu.SemaphoreType.DMA,
            pltpu.SemaphoreType.DMA,
        ],
    )
    p = run(x0, x1, unique_pos_w, layer_pos_w, rot_token_w)
    return jnp.transpose(p, (1, 0, 2))
